# baseline (speedup 1.0000x reference)
import jax
import jax.numpy as jnp
from jax import lax
from jax.experimental import pallas as pl
from jax.experimental.pallas import tpu as pltpu
from jax.experimental.pallas import tpu_sc as plsc

NUM_USER = 1000000
NUM_ITEM = 100000
NUM_TIME = 200
D = 32
BATCH = 16384

NC = 2
NS = 16
LANES = 16
DPC = D // NC
BPW = BATCH // NS
NBL = BPW // 128
NCH = DPC * NBL
LAG = 8

UPW = (NUM_USER // (128 * NS)) * 128
UREM0 = UPW * NS
UREM = (NUM_USER // 128) * 128 - UREM0
UFLAT0 = UREM0 + UREM
UTAIL = NUM_USER - UFLAT0
USTRIDE = NUM_USER + 64


def _body(u_hbm, v_hbm, t_hbm, ut_hbm, i_hbm, j_hbm, k_hbm, out_hbm,
          scr, iv, jv, kv, idxu, idxv, ubuf, vbuf, tbuf, outv, sem):
  c = lax.axis_index("c")
  s = lax.axis_index("s")
  nbase = s * BPW
  dbase = c * DPC

  for d_loc in range(DPC):
    row = dbase + d_loc
    src0 = pl.multiple_of(s * UPW, 128)
    pltpu.sync_copy(u_hbm.at[row, pl.ds(src0, UPW)],
                    scr.at[pl.ds(row * USTRIDE + src0, UPW)])

  @pl.when(s == 0)
  def _():
    for d_loc in range(DPC):
      row = dbase + d_loc
      pltpu.sync_copy(u_hbm.at[row, pl.ds(UREM0, UREM)],
                      scr.at[pl.ds(row * USTRIDE + UREM0, UREM)])
      pltpu.sync_copy(ut_hbm.at[pl.ds(row * 128, 128)],
                      scr.at[pl.ds(row * USTRIDE + UFLAT0, 128)])

  plsc.subcore_barrier()

  pltpu.sync_copy(i_hbm.at[pl.ds(nbase, BPW)], iv)
  pltpu.sync_copy(j_hbm.at[pl.ds(nbase, BPW)], jv)
  pltpu.sync_copy(k_hbm.at[pl.ds(nbase, BPW)], kv)
  pltpu.sync_copy(t_hbm, tbuf)

  def build_idx(r, carry):
    d_loc = r // NBL
    b = r % NBL
    uoff = (dbase + d_loc) * USTRIDE
    voff = (dbase + d_loc) * NUM_ITEM
    for q in range(128 // LANES):
      src = pl.ds(b * 128 + q * LANES, LANES)
      dst = pl.ds(r * 128 + q * LANES, LANES)
      idxu[dst] = iv[src] + uoff
      idxv[dst] = jv[src] + voff
    return carry

  lax.fori_loop(0, NCH, build_idx, 0)

  def fire(ch):
    sl = pl.ds(ch * 128, 128)
    pltpu.async_copy(scr.at[idxu.at[sl]], ubuf.at[sl], sem)
    pltpu.async_copy(v_hbm.at[idxv.at[sl]], vbuf.at[sl], sem)

  def wait_for(ch):
    sl = pl.ds(ch * 128, 128)
    pltpu.make_async_copy(scr.at[idxu.at[sl]], ubuf.at[sl], sem).wait()
    pltpu.make_async_copy(v_hbm.at[idxv.at[sl]], vbuf.at[sl], sem).wait()

  def gstep(ch, carry):
    fire(ch)
    @pl.when(ch >= LAG)
    def _():
      wait_for(ch - LAG)
    return carry

  lax.fori_loop(0, NCH, gstep, 0)

  def dstep(ch, carry):
    wait_for(ch)
    return carry

  lax.fori_loop(NCH - LAG, NCH, dstep, 0)

  def compute(nv_i, carry):
    nb16 = nv_i * LANES
    kvv = kv[pl.ds(nb16, LANES)]
    acc = jnp.zeros((LANES,), jnp.float32)
    for d_loc in range(DPC):
      uu = ubuf[pl.ds(d_loc * BPW + nb16, LANES)]
      vv = vbuf[pl.ds(d_loc * BPW + nb16, LANES)]
      tt = plsc.load_gather(tbuf, [kvv + (dbase + d_loc) * NUM_TIME])
      acc = acc + uu * vv * tt
    outv[pl.ds(nb16, LANES)] = acc
    return carry

  lax.fori_loop(0, BPW // LANES, compute, 0)

  pltpu.sync_copy(outv, out_hbm.at[pl.ds(c * BATCH + nbase, BPW)])


@jax.jit
def _run(user_embeddings, item_embeddings, time_embeddings,
         i_input, j_input, k_input):
  mesh = plsc.VectorSubcoreMesh(core_axis_name="c", subcore_axis_name="s")
  f = pl.kernel(
      _body,
      out_type=jax.ShapeDtypeStruct((NC * BATCH,), jnp.float32),
      mesh=mesh,
      compiler_params=pltpu.CompilerParams(
          needs_layout_passes=False, use_tc_tiling_on_sc=True),
      scratch_types=[
          pltpu.HBM((D * USTRIDE,), jnp.float32),
          pltpu.VMEM((BPW,), jnp.int32),
          pltpu.VMEM((BPW,), jnp.int32),
          pltpu.VMEM((BPW,), jnp.int32),
          pltpu.VMEM((NCH * 128,), jnp.int32),
          pltpu.VMEM((NCH * 128,), jnp.int32),
          pltpu.VMEM((BPW * DPC,), jnp.float32),
          pltpu.VMEM((BPW * DPC,), jnp.float32),
          pltpu.VMEM((NUM_TIME * D,), jnp.float32),
          pltpu.VMEM((BPW,), jnp.float32),
          pltpu.SemaphoreType.DMA,
      ],
  )
  u_tail = jnp.pad(user_embeddings[UFLAT0:].T, ((0, 0), (0, 128 - UTAIL)))
  u_tail = u_tail.reshape(-1)
  o2 = f(user_embeddings.T, item_embeddings.T.reshape(-1),
         time_embeddings.T.reshape(-1), u_tail,
         i_input, j_input, k_input)
  return o2[:BATCH] + o2[BATCH:]


def kernel(user_embeddings, item_embeddings, time_embeddings,
           i_input, j_input, k_input):
  return _run(user_embeddings, item_embeddings, time_embeddings,
              i_input.astype(jnp.int32), j_input.astype(jnp.int32),
              k_input.astype(jnp.int32))


# speedup vs baseline: 15.5549x; 15.5549x over previous
import jax
import jax.numpy as jnp
from jax import lax
from jax.experimental import pallas as pl
from jax.experimental.pallas import tpu as pltpu
from jax.experimental.pallas import tpu_sc as plsc

NUM_USER = 1000000
NUM_ITEM = 100000
NUM_TIME = 200
D = 32
BATCH = 16384

NC = 2
NS = 16
LANES = 16
DPC = D // NC
BPW = BATCH // NS
NBL = BPW // 128
NCH = DPC * NBL
LAG = 8

BW = 2048
BW8 = 8 * BW
SHB = BW.bit_length() - 1
SHW = BW8.bit_length() - 1
NBLK = NUM_USER // BW + 1
LASTW = NUM_USER - (NBLK - 1) * BW
LASTA = (LASTW // 128) * 128
UTAIL = LASTW - LASTA
TPW = NBLK // NS + 1


def _body(u_hbm, v_hbm, t_hbm, ut_hbm, i_hbm, j_hbm, k_hbm, out_hbm,
          scr, sb0, sb1, ptail, iv, jv, kv, idxu, idxv, ubuf, vbuf,
          tbuf, outv, sem):
  c = lax.axis_index("c")
  s = lax.axis_index("s")
  nbase = s * BPW
  dbase = c * DPC

  for g_loc in range(2):
    g = c * 2 + g_loc
    row0 = g * 8
    gbase = g * NBLK * BW8

    def wait_writes(sb, blk):
      for r in range(8):
        pltpu.make_async_copy(
            sb.at[r], scr.at[pl.ds(gbase + blk * BW8 + r * BW, BW)],
            sem).wait()

    def do_turn(sb, t):
      blk = s + t * NS
      @pl.when(t >= 2)
      def _():
        wait_writes(sb, blk - 2 * NS)
      pltpu.sync_copy(u_hbm.at[pl.ds(row0, 8), pl.ds(blk * BW, BW)], sb)
      for r in range(8):
        pltpu.async_copy(
            sb.at[r], scr.at[pl.ds(gbase + blk * BW8 + r * BW, BW)], sem)

    def pair(m, carry):
      do_turn(sb0, 2 * m)
      do_turn(sb1, 2 * m + 1)
      return carry

    lax.fori_loop(0, (TPW - 1) // 2, pair, 0)
    wait_writes(sb0, s + (TPW - 3) * NS)
    wait_writes(sb1, s + (TPW - 2) * NS)

    blk = s + (TPW - 1) * NS
    @pl.when(blk < NBLK - 1)
    def _():
      pltpu.sync_copy(u_hbm.at[pl.ds(row0, 8), pl.ds(blk * BW, BW)], sb0)
      for r in range(8):
        pltpu.sync_copy(
            sb0.at[r], scr.at[pl.ds(gbase + blk * BW8 + r * BW, BW)])
    @pl.when(blk == NBLK - 1)
    def _():
      lb = (NBLK - 1) * BW
      pltpu.sync_copy(u_hbm.at[pl.ds(row0, 8), pl.ds(lb, LASTA)], ptail)
      for r in range(8):
        pltpu.sync_copy(
            ptail.at[r],
            scr.at[pl.ds(gbase + blk * BW8 + r * BW, LASTA)])
        pltpu.sync_copy(
            ut_hbm.at[pl.ds((row0 + r) * 128, 128)],
            scr.at[pl.ds(gbase + blk * BW8 + r * BW + LASTA, 128)])

  plsc.subcore_barrier()

  pltpu.sync_copy(i_hbm.at[pl.ds(nbase, BPW)], iv)
  pltpu.sync_copy(j_hbm.at[pl.ds(nbase, BPW)], jv)
  pltpu.sync_copy(k_hbm.at[pl.ds(nbase, BPW)], kv)
  pltpu.sync_copy(t_hbm, tbuf)

  def build_idx(rr, carry):
    d_loc = rr // NBL
    b = rr % NBL
    g = c * 2 + d_loc // 8
    r = d_loc % 8
    ubase = g * NBLK + 0
    voff = (dbase + d_loc) * NUM_ITEM
    for q in range(128 // LANES):
      src = pl.ds(b * 128 + q * LANES, LANES)
      dst = pl.ds(rr * 128 + q * LANES, LANES)
      ivv = iv[src]
      idxu[dst] = (((ubase + (ivv >> SHB)) << SHW) + (r << SHB)
                   + (ivv & (BW - 1)))
      idxv[dst] = jv[src] + voff
    return carry

  lax.fori_loop(0, NCH, build_idx, 0)

  def fire(ch):
    sl = pl.ds(ch * 128, 128)
    pltpu.async_copy(scr.at[idxu.at[sl]], ubuf.at[sl], sem)
    pltpu.async_copy(v_hbm.at[idxv.at[sl]], vbuf.at[sl], sem)

  def wait_for(ch):
    sl = pl.ds(ch * 128, 128)
    pltpu.make_async_copy(scr.at[idxu.at[sl]], ubuf.at[sl], sem).wait()
    pltpu.make_async_copy(v_hbm.at[idxv.at[sl]], vbuf.at[sl], sem).wait()

  def gstep(ch, carry):
    fire(ch)
    @pl.when(ch >= LAG)
    def _():
      wait_for(ch - LAG)
    return carry

  lax.fori_loop(0, NCH, gstep, 0)

  def dstep(ch, carry):
    wait_for(ch)
    return carry

  lax.fori_loop(NCH - LAG, NCH, dstep, 0)

  def compute(nv_i, carry):
    nb16 = nv_i * LANES
    kvv = kv[pl.ds(nb16, LANES)]
    acc = jnp.zeros((LANES,), jnp.float32)
    for d_loc in range(DPC):
      uu = ubuf[pl.ds(d_loc * BPW + nb16, LANES)]
      vv = vbuf[pl.ds(d_loc * BPW + nb16, LANES)]
      tt = plsc.load_gather(tbuf, [kvv + (dbase + d_loc) * NUM_TIME])
      acc = acc + uu * vv * tt
    outv[pl.ds(nb16, LANES)] = acc
    return carry

  lax.fori_loop(0, BPW // LANES, compute, 0)

  pltpu.sync_copy(outv, out_hbm.at[pl.ds(c * BATCH + nbase, BPW)])


@jax.jit
def _run(user_embeddings, item_embeddings, time_embeddings,
         i_input, j_input, k_input):
  mesh = plsc.VectorSubcoreMesh(core_axis_name="c", subcore_axis_name="s")
  f = pl.kernel(
      _body,
      out_type=jax.ShapeDtypeStruct((NC * BATCH,), jnp.float32),
      mesh=mesh,
      compiler_params=pltpu.CompilerParams(
          needs_layout_passes=False, use_tc_tiling_on_sc=True),
      scratch_types=[
          pltpu.HBM((4 * NBLK * BW8,), jnp.float32),
          pltpu.VMEM((8, BW), jnp.float32),
          pltpu.VMEM((8, BW), jnp.float32),
          pltpu.VMEM((8, LASTA), jnp.float32),
          pltpu.VMEM((BPW,), jnp.int32),
          pltpu.VMEM((BPW,), jnp.int32),
          pltpu.VMEM((BPW,), jnp.int32),
          pltpu.VMEM((NCH * 128,), jnp.int32),
          pltpu.VMEM((NCH * 128,), jnp.int32),
          pltpu.VMEM((BPW * DPC,), jnp.float32),
          pltpu.VMEM((BPW * DPC,), jnp.float32),
          pltpu.VMEM((NUM_TIME * D,), jnp.float32),
          pltpu.VMEM((BPW,), jnp.float32),
          pltpu.SemaphoreType.DMA,
      ],
  )
  lastu = (NBLK - 1) * BW + LASTA
  u_tail = jnp.pad(user_embeddings[lastu:].T, ((0, 0), (0, 128 - UTAIL)))
  u_tail = u_tail.reshape(-1)
  o2 = f(user_embeddings.T, item_embeddings.T.reshape(-1),
         time_embeddings.T.reshape(-1), u_tail,
         i_input, j_input, k_input)
  return o2[:BATCH] + o2[BATCH:]


def kernel(user_embeddings, item_embeddings, time_embeddings,
           i_input, j_input, k_input):
  return _run(user_embeddings, item_embeddings, time_embeddings,
              i_input.astype(jnp.int32), j_input.astype(jnp.int32),
              k_input.astype(jnp.int32))


# speedup vs baseline: 18.4073x; 1.1834x over previous
import jax
import jax.numpy as jnp
from jax import lax
from jax.experimental import pallas as pl
from jax.experimental.pallas import tpu as pltpu
from jax.experimental.pallas import tpu_sc as plsc

NUM_USER = 1000000
NUM_ITEM = 100000
NUM_TIME = 200
D = 32
BATCH = 16384

NC = 2
NS = 16
LANES = 16
DPC = D // NC
BPW = BATCH // NS
NBL = BPW // 128
NCH = DPC * NBL
LAG = 8

BW = 2048
BW8 = 8 * BW
SHB = BW.bit_length() - 1
SHW = BW8.bit_length() - 1
NBLK = NUM_USER // BW + 1
LASTW = NUM_USER - (NBLK - 1) * BW
LASTA = (LASTW // 128) * 128
UTAIL = LASTW - LASTA
TPW = NBLK // NS + 1


def _body(u_hbm, v_hbm, t_hbm, ut_hbm, i_hbm, j_hbm, k_hbm, out_hbm,
          scr, sb0, sb1, ptail, iv, jv, kv, idxu, idxv, ubuf, vbuf,
          tbuf, outv, sem, sem_r):
  c = lax.axis_index("c")
  s = lax.axis_index("s")
  nbase = s * BPW
  dbase = c * DPC

  for g_loc in range(2):
    g = c * 2 + g_loc
    row0 = g * 8
    gbase = g * NBLK * BW8

    def wait_writes(sb, blk):
      for r in range(8):
        pltpu.make_async_copy(
            sb.at[r], scr.at[pl.ds(gbase + blk * BW8 + r * BW, BW)],
            sem).wait()

    def slab_src(blk):
      return u_hbm.at[pl.ds(row0, 8), pl.ds(blk * BW, BW)]

    def do_turn(sb, t):
      blk = s + t * NS
      pltpu.make_async_copy(slab_src(blk), sb, sem_r).wait()
      for r in range(8):
        pltpu.async_copy(
            sb.at[r], scr.at[pl.ds(gbase + blk * BW8 + r * BW, BW)], sem)

    def next_read(sb, t):
      blk = s + t * NS
      @pl.when(t + 2 <= TPW - 2)
      def _():
        wait_writes(sb, blk)
        pltpu.async_copy(slab_src(blk + 2 * NS), sb, sem_r)

    def pair(m, carry):
      do_turn(sb0, 2 * m)
      do_turn(sb1, 2 * m + 1)
      next_read(sb0, 2 * m)
      next_read(sb1, 2 * m + 1)
      return carry

    pltpu.async_copy(slab_src(s), sb0, sem_r)
    pltpu.async_copy(slab_src(s + NS), sb1, sem_r)
    lax.fori_loop(0, (TPW - 1) // 2, pair, 0)
    wait_writes(sb0, s + (TPW - 3) * NS)
    wait_writes(sb1, s + (TPW - 2) * NS)

    blk = s + (TPW - 1) * NS
    @pl.when(blk < NBLK - 1)
    def _():
      pltpu.sync_copy(u_hbm.at[pl.ds(row0, 8), pl.ds(blk * BW, BW)], sb0)
      for r in range(8):
        pltpu.sync_copy(
            sb0.at[r], scr.at[pl.ds(gbase + blk * BW8 + r * BW, BW)])
    @pl.when(blk == NBLK - 1)
    def _():
      lb = (NBLK - 1) * BW
      pltpu.sync_copy(u_hbm.at[pl.ds(row0, 8), pl.ds(lb, LASTA)], ptail)
      for r in range(8):
        pltpu.sync_copy(
            ptail.at[r],
            scr.at[pl.ds(gbase + blk * BW8 + r * BW, LASTA)])
        pltpu.sync_copy(
            ut_hbm.at[pl.ds((row0 + r) * 128, 128)],
            scr.at[pl.ds(gbase + blk * BW8 + r * BW + LASTA, 128)])

  plsc.subcore_barrier()

  pltpu.sync_copy(i_hbm.at[pl.ds(nbase, BPW)], iv)
  pltpu.sync_copy(j_hbm.at[pl.ds(nbase, BPW)], jv)
  pltpu.sync_copy(k_hbm.at[pl.ds(nbase, BPW)], kv)
  pltpu.sync_copy(t_hbm, tbuf)

  def build_idx(rr, carry):
    d_loc = rr // NBL
    b = rr % NBL
    g = c * 2 + d_loc // 8
    r = d_loc % 8
    ubase = g * NBLK + 0
    voff = (dbase + d_loc) * NUM_ITEM
    for q in range(128 // LANES):
      src = pl.ds(b * 128 + q * LANES, LANES)
      dst = pl.ds(rr * 128 + q * LANES, LANES)
      ivv = iv[src]
      idxu[dst] = (((ubase + (ivv >> SHB)) << SHW) + (r << SHB)
                   + (ivv & (BW - 1)))
      idxv[dst] = jv[src] + voff
    return carry

  lax.fori_loop(0, NCH, build_idx, 0)

  def fire(ch):
    sl = pl.ds(ch * 128, 128)
    pltpu.async_copy(scr.at[idxu.at[sl]], ubuf.at[sl], sem)
    pltpu.async_copy(v_hbm.at[idxv.at[sl]], vbuf.at[sl], sem)

  def wait_for(ch):
    sl = pl.ds(ch * 128, 128)
    pltpu.make_async_copy(scr.at[idxu.at[sl]], ubuf.at[sl], sem).wait()
    pltpu.make_async_copy(v_hbm.at[idxv.at[sl]], vbuf.at[sl], sem).wait()

  def gstep(ch, carry):
    fire(ch)
    @pl.when(ch >= LAG)
    def _():
      wait_for(ch - LAG)
    return carry

  lax.fori_loop(0, NCH, gstep, 0)

  def dstep(ch, carry):
    wait_for(ch)
    return carry

  lax.fori_loop(NCH - LAG, NCH, dstep, 0)

  def compute(nv_i, carry):
    nb16 = nv_i * LANES
    kvv = kv[pl.ds(nb16, LANES)]
    acc = jnp.zeros((LANES,), jnp.float32)
    for d_loc in range(DPC):
      uu = ubuf[pl.ds(d_loc * BPW + nb16, LANES)]
      vv = vbuf[pl.ds(d_loc * BPW + nb16, LANES)]
      tt = plsc.load_gather(tbuf, [kvv + (dbase + d_loc) * NUM_TIME])
      acc = acc + uu * vv * tt
    outv[pl.ds(nb16, LANES)] = acc
    return carry

  lax.fori_loop(0, BPW // LANES, compute, 0)

  pltpu.sync_copy(outv, out_hbm.at[pl.ds(c * BATCH + nbase, BPW)])


@jax.jit
def _run(user_embeddings, item_embeddings, time_embeddings,
         i_input, j_input, k_input):
  mesh = plsc.VectorSubcoreMesh(core_axis_name="c", subcore_axis_name="s")
  f = pl.kernel(
      _body,
      out_type=jax.ShapeDtypeStruct((NC * BATCH,), jnp.float32),
      mesh=mesh,
      compiler_params=pltpu.CompilerParams(
          needs_layout_passes=False, use_tc_tiling_on_sc=True),
      scratch_types=[
          pltpu.HBM((4 * NBLK * BW8,), jnp.float32),
          pltpu.VMEM((8, BW), jnp.float32),
          pltpu.VMEM((8, BW), jnp.float32),
          pltpu.VMEM((8, LASTA), jnp.float32),
          pltpu.VMEM((BPW,), jnp.int32),
          pltpu.VMEM((BPW,), jnp.int32),
          pltpu.VMEM((BPW,), jnp.int32),
          pltpu.VMEM((NCH * 128,), jnp.int32),
          pltpu.VMEM((NCH * 128,), jnp.int32),
          pltpu.VMEM((BPW * DPC,), jnp.float32),
          pltpu.VMEM((BPW * DPC,), jnp.float32),
          pltpu.VMEM((NUM_TIME * D,), jnp.float32),
          pltpu.VMEM((BPW,), jnp.float32),
          pltpu.SemaphoreType.DMA,
          pltpu.SemaphoreType.DMA,
      ],
  )
  lastu = (NBLK - 1) * BW + LASTA
  u_tail = jnp.pad(user_embeddings[lastu:].T, ((0, 0), (0, 128 - UTAIL)))
  u_tail = u_tail.reshape(-1)
  o2 = f(user_embeddings.T, item_embeddings.T.reshape(-1),
         time_embeddings.T.reshape(-1), u_tail,
         i_input, j_input, k_input)
  return o2[:BATCH] + o2[BATCH:]


def kernel(user_embeddings, item_embeddings, time_embeddings,
           i_input, j_input, k_input):
  return _run(user_embeddings, item_embeddings, time_embeddings,
              i_input.astype(jnp.int32), j_input.astype(jnp.int32),
              k_input.astype(jnp.int32))
